# baseline (speedup 1.0000x reference)
import jax
import jax.numpy as jnp
from jax.experimental import pallas as pl
from jax.experimental.pallas import tpu as pltpu

_MB = 16
_MAIN = 192
_TAIL = 8


def _main_kernel(x_ref, posb_ref, wt_ref, o_ref):
    wt = wt_ref[...]
    base = pl.program_id(0) * _MB
    for t in range(_MB):
        acc = jax.lax.dot_general(
            wt, x_ref[t], (((1,), (0,)), ((), ())),
            preferred_element_type=jnp.float32)
        o_ref[t] = acc + posb_ref[base + t][:, None]


def _tail_kernel(prev_ref, x_ref, posb_ref, wt_ref, o_ref):
    del prev_ref
    wt = wt_ref[...]
    for t in range(_TAIL):
        acc = jax.lax.dot_general(
            wt, x_ref[t], (((1,), (0,)), ((), ())),
            preferred_element_type=jnp.float32)
        o_ref[t] = acc + posb_ref[_MAIN + t][:, None]


def kernel(x, pos_table, W, b):
    B, L, D = x.shape
    xt = jnp.transpose(x, (1, 2, 0))
    posb = pos_table + b[None, :]
    wt = W.T

    out1 = pl.pallas_call(
        _main_kernel,
        grid=(_MAIN // _MB,),
        in_specs=[
            pl.BlockSpec((_MB, D, B), lambda i: (i, 0, 0)),
            pl.BlockSpec((L, D), lambda i: (0, 0)),
            pl.BlockSpec((D, D), lambda i: (0, 0)),
        ],
        out_specs=pl.BlockSpec((_MB, D, B), lambda i: (i, 0, 0)),
        out_shape=jax.ShapeDtypeStruct((L, D, B), x.dtype),
    )(xt, posb, wt)

    out = pl.pallas_call(
        _tail_kernel,
        grid=(1,),
        in_specs=[
            pl.BlockSpec(memory_space=pltpu.MemorySpace.HBM),
            pl.BlockSpec((_TAIL, D, B), lambda i: (_MAIN // _TAIL, 0, 0)),
            pl.BlockSpec((L, D), lambda i: (0, 0)),
            pl.BlockSpec((D, D), lambda i: (0, 0)),
        ],
        out_specs=pl.BlockSpec((_TAIL, D, B),
                               lambda i: (_MAIN // _TAIL, 0, 0)),
        out_shape=jax.ShapeDtypeStruct((L, D, B), x.dtype),
        input_output_aliases={0: 0},
    )(out1, xt, posb, wt)
    return jnp.transpose(out, (2, 0, 1))


# speedup vs baseline: 1.0383x; 1.0383x over previous
import jax
import jax.numpy as jnp
from jax.experimental import pallas as pl
from jax.experimental.pallas import tpu as pltpu

_MB = 20
_DIN = 3
_DOUT = 2


def _embed_kernel(x_hbm, posb_ref, wt_ref, o_hbm, xbuf, obuf, in_sems,
                  out_sems):
    i = pl.program_id(0)
    n = pl.num_programs(0)

    def in_copy(c, s):
        return pltpu.make_async_copy(
            x_hbm.at[pl.ds(c * _MB, _MB)], xbuf.at[s], in_sems.at[s])

    def out_copy(c, s):
        return pltpu.make_async_copy(
            obuf.at[s], o_hbm.at[pl.ds(c * _MB, _MB)], out_sems.at[s])

    @pl.when(i == 0)
    def _():
        for d in range(_DIN - 1):
            in_copy(d, d).start()

    @pl.when(i + _DIN - 1 < n)
    def _():
        c = i + _DIN - 1
        in_copy(c, jax.lax.rem(c, _DIN)).start()

    islot = jax.lax.rem(i, _DIN)
    oslot = jax.lax.rem(i, _DOUT)
    in_copy(i, islot).wait()

    @pl.when(i >= _DOUT)
    def _():
        out_copy(i - _DOUT, oslot).wait()

    wt = wt_ref[...]
    base = i * _MB
    for t in range(_MB):
        acc = jax.lax.dot_general(
            wt, xbuf[islot, t], (((1,), (0,)), ((), ())),
            preferred_element_type=jnp.float32)
        obuf[oslot, t] = acc + posb_ref[base + t][:, None]

    out_copy(i, oslot).start()

    @pl.when(i == n - 1)
    def _():
        for d in range(_DOUT):
            c = n - 1 - d
            if c >= 0:
                out_copy(c, c % _DOUT).wait()


def kernel(x, pos_table, W, b):
    B, L, D = x.shape
    xt = jnp.transpose(x, (1, 2, 0))
    posb = pos_table + b[None, :]
    wt = W.T

    out = pl.pallas_call(
        _embed_kernel,
        grid=(L // _MB,),
        in_specs=[
            pl.BlockSpec(memory_space=pltpu.MemorySpace.HBM),
            pl.BlockSpec((L, D), lambda i: (0, 0)),
            pl.BlockSpec((D, D), lambda i: (0, 0)),
        ],
        out_specs=pl.BlockSpec(memory_space=pltpu.MemorySpace.HBM),
        out_shape=jax.ShapeDtypeStruct((L, D, B), x.dtype),
        scratch_shapes=[
            pltpu.VMEM((_DIN, _MB, D, B), jnp.float32),
            pltpu.VMEM((_DOUT, _MB, D, B), jnp.float32),
            pltpu.SemaphoreType.DMA((_DIN,)),
            pltpu.SemaphoreType.DMA((_DOUT,)),
        ],
    )(xt, posb, wt)
    return jnp.transpose(out, (2, 0, 1))


# speedup vs baseline: 1.0487x; 1.0101x over previous
import jax
import jax.numpy as jnp
from jax.experimental import pallas as pl

_MB = 25


def _embed_kernel(x_ref, posb_ref, wt_ref, o_ref):
    wt = wt_ref[...]
    base = pl.program_id(0) * _MB
    for t in range(_MB):
        acc = jax.lax.dot_general(
            wt, x_ref[t], (((1,), (0,)), ((), ())),
            preferred_element_type=jnp.float32)
        o_ref[t] = acc + posb_ref[base + t][:, None]


def kernel(x, pos_table, W, b):
    B, L, D = x.shape
    xt = jnp.transpose(x, (1, 2, 0))
    posb = pos_table + b[None, :]
    wt = W.T

    out = pl.pallas_call(
        _embed_kernel,
        grid=(L // _MB,),
        in_specs=[
            pl.BlockSpec((_MB, D, B), lambda i: (i, 0, 0)),
            pl.BlockSpec((L, D), lambda i: (0, 0)),
            pl.BlockSpec((D, D), lambda i: (0, 0)),
        ],
        out_specs=pl.BlockSpec((_MB, D, B), lambda i: (i, 0, 0)),
        out_shape=jax.ShapeDtypeStruct((L, D, B), x.dtype),
    )(xt, posb, wt)
    return jnp.transpose(out, (2, 0, 1))
